# baseline (speedup 1.0000x reference)
import functools

import jax
import jax.numpy as jnp
from jax import lax
from jax.experimental import pallas as pl
from jax.experimental.pallas import tpu as pltpu
from jax.experimental.pallas import tpu_sc as plsc

N = 10000
D = 128
H = 256
HH = 128
E = 320000
G = 64
NC = 2
NS = 16
L = 16

NP = 10240
EP = 327680
KE = 128
RB = 1000

_mesh = functools.partial(
    plsc.VectorSubcoreMesh,
    core_axis_name="c", subcore_axis_name="s", num_cores=NC, num_subcores=NS)


def _emb_gather(emb, mapping2):
    BPW = NP // (NC * NS)
    CH = 80

    def body(emb_hbm, idx_hbm, out_hbm, idx_v, rows_v, sem):
        wid = lax.axis_index("s") * NC + lax.axis_index("c")
        base = wid * BPW
        pltpu.sync_copy(idx_hbm.at[pl.ds(wid * (BPW // CH), BPW // CH)], idx_v)
        for q in range(BPW // CH):
            pltpu.async_copy(emb_hbm.at[idx_v.at[q]],
                             rows_v.at[pl.ds(q * CH, CH)], sem).wait()
        pltpu.sync_copy(rows_v, out_hbm.at[pl.ds(base, BPW)])

    return pl.kernel(
        body,
        out_type=jax.ShapeDtypeStruct((NP, D), jnp.float32),
        mesh=_mesh(),
        scratch_types=[
            pltpu.VMEM((BPW // CH, CH), jnp.int32),
            pltpu.VMEM((BPW, D), jnp.float32),
            pltpu.SemaphoreType.DMA,
        ],
    )(emb, mapping2)


def _deg(colp, wp):
    EPW = EP // (NC * NS)
    ZR = NP // NS

    def body(col_hbm, w_hbm, out_hbm, acc_sh, colv, wv, cidx, zbuf, sem):
        c = lax.axis_index("c")
        s = lax.axis_index("s")
        wid = c * NS + s
        pltpu.sync_copy(col_hbm.at[pl.ds(wid * EPW, EPW)], colv)
        pltpu.sync_copy(w_hbm.at[pl.ds(wid * EPW, EPW)], wv)
        zeros = jnp.zeros((L,), jnp.float32)
        for r in range(L):
            zbuf[pl.ds(r * L, L)] = zeros
        def zloop(q, _):
            pltpu.sync_copy(zbuf, acc_sh.at[pl.ds(s * ZR + q * L * L, L * L)])
            return 0
        lax.fori_loop(0, ZR // (L * L), zloop, 0)
        plsc.subcore_barrier()

        def chunk(k, _):
            for j in range(KE // L):
                cidx[pl.ds(j * L, L)] = colv[pl.ds(k * KE + j * L, L)]
            pltpu.async_copy(wv.at[pl.ds(k * KE, KE)], acc_sh.at[cidx],
                             sem, add=True).wait()
            return 0
        lax.fori_loop(0, EPW // KE, chunk, 0)
        plsc.subcore_barrier()
        pltpu.sync_copy(acc_sh.at[pl.ds(s * ZR, ZR)],
                        out_hbm.at[pl.ds(c * NP + s * ZR, ZR)])

    return pl.kernel(
        body,
        out_type=jax.ShapeDtypeStruct((NC * NP,), jnp.float32),
        mesh=_mesh(),
        scratch_types=[
            pltpu.VMEM_SHARED((NP,), jnp.float32),
            pltpu.VMEM((EPW,), jnp.int32),
            pltpu.VMEM((EPW,), jnp.float32),
            pltpu.VMEM((KE,), jnp.int32),
            pltpu.VMEM((L * L,), jnp.float32),
            pltpu.SemaphoreType.DMA,
        ],
    )(colp, wp)


def _agg(hlpf, eidx, ewf):
    KB = 64
    BPT = (EP // NS) // KB
    ZR = NP // NS
    NSET = 4

    def body(hl_hbm, ep_hbm, ew_hbm, out_hbm, acc_sh,
             eb0, eb1, eb2, eb3, vb0, vb1, vb2, vb3,
             gi0, gi1, gi2, gi3, ci0, ci1, ci2, ci3,
             wb0, wb1, wb2, wb3, rb0, rb1, rb2, rb3,
             es0, es1, es2, es3, gs0, gs1, gs2, gs3, ss0, ss1, ss2, ss3):
        ebuf = [eb0, eb1, eb2, eb3]
        vbuf = [vb0, vb1, vb2, vb3]
        gidx = [gi0, gi1, gi2, gi3]
        cidx = [ci0, ci1, ci2, ci3]
        wblk = [wb0, wb1, wb2, wb3]
        rbuf = [rb0, rb1, rb2, rb3]
        esem = [es0, es1, es2, es3]
        gsem = [gs0, gs1, gs2, gs3]
        ssem = [ss0, ss1, ss2, ss3]
        c = lax.axis_index("c")
        s = lax.axis_index("s")
        zeros = jnp.zeros((L,), jnp.float32)
        for r in range(L):
            for q in range(HH // L):
                rb0[r, pl.ds(q * L, L)] = zeros
        def zloop(q, _):
            pltpu.sync_copy(rb0.at[pl.ds(0, L)],
                            acc_sh.at[pl.ds(s * ZR + q * L, L)])
            return 0
        lax.fori_loop(0, ZR // L, zloop, 0)
        plsc.subcore_barrier()

        coff = c * N
        gb0 = s * BPT

        def stage(m, gb):
            pltpu.async_copy(ep_hbm.at[pl.ds(gb * (2 * KB), 2 * KB)],
                             ebuf[m], esem[m])
            pltpu.async_copy(ew_hbm.at[pl.ds(gb * KB, KB)],
                             vbuf[m], esem[m])

        def wait_stage(m):
            pltpu.make_async_copy(ep_hbm.at[pl.ds(0, 2 * KB)],
                                  ebuf[m], esem[m]).wait()
            pltpu.make_async_copy(ew_hbm.at[pl.ds(0, KB)],
                                  vbuf[m], esem[m]).wait()

        def build(m):
            for j in range(KB // L):
                o = j * L
                cidx[m][pl.ds(o, L)] = ebuf[m][pl.ds(KB + o, L)]
                gidx[m][pl.ds(o, L)] = ebuf[m][pl.ds(o, L)] + coff
                wblk[m][pl.ds(o, L)] = vbuf[m][pl.ds(o, L)]

        def gather(m):
            pltpu.async_copy(hl_hbm.at[gidx[m]], rbuf[m], gsem[m])

        def wait_gather(m):
            pltpu.make_async_copy(hl_hbm.at[gidx[m]], rbuf[m], gsem[m]).wait()

        def scatter(m):
            pltpu.async_copy(rbuf[m], acc_sh.at[cidx[m]], ssem[m], add=True)

        def wait_scatter(m):
            pltpu.make_async_copy(rbuf[m], acc_sh.at[cidx[m]], ssem[m]).wait()

        def scale(m):
            rb = rbuf[m]
            wb = wblk[m]
            def sj(j, _):
                wvv = wb[pl.ds(j * L, L)]
                for t in range(L):
                    ns = jnp.broadcast_to(wvv[t], (L,))
                    e = j * L + t
                    for q in range(HH // L):
                        rb[e, pl.ds(q * L, L)] = rb[e, pl.ds(q * L, L)] * ns
                return 0
            lax.fori_loop(0, KB // L, sj, 0)

        for m in range(NSET):
            stage(m, gb0 + m)
        trash = jnp.full((L,), NP - 1, jnp.int32)
        for m in (2, 3):
            for j in range(KB // L):
                cidx[m][pl.ds(j * L, L)] = trash
            scatter(m)
        for m in (0, 1):
            wait_stage(m)
            build(m)
            gather(m)

        NIT = BPT // NSET - 1

        def slot(m, b):
            m2 = (m + 2) % NSET
            wait_scatter(m2)
            wait_stage(m2)
            build(m2)
            gather(m2)
            stage(m, gb0 + b + NSET)
            wait_gather(m)
            scale(m)
            scatter(m)

        def it(i, _):
            b = i * NSET
            for m in range(NSET):
                slot(m, b + m)
            return 0
        lax.fori_loop(0, NIT, it, 0)

        for k in range(2):
            m = k
            m2 = (m + 2) % NSET
            wait_scatter(m2)
            wait_stage(m2)
            build(m2)
            gather(m2)
            wait_gather(m)
            scale(m)
            scatter(m)
        for m in (2, 3):
            wait_gather(m)
            scale(m)
            scatter(m)
        for m in range(NSET):
            wait_scatter(m)

        plsc.subcore_barrier()
        pltpu.sync_copy(acc_sh.at[pl.ds(s * ZR, ZR)],
                        out_hbm.at[pl.ds(c * NP + s * ZR, ZR)])

    KB = 64
    return pl.kernel(
        body,
        out_type=jax.ShapeDtypeStruct((NC * NP, HH), jnp.float32),
        mesh=_mesh(),
        scratch_types=(
            [pltpu.VMEM_SHARED((NP, HH), jnp.float32)]
            + [pltpu.VMEM((2 * KB,), jnp.int32) for _ in range(4)]
            + [pltpu.VMEM((KB,), jnp.float32) for _ in range(4)]
            + [pltpu.VMEM((KB,), jnp.int32) for _ in range(4)]
            + [pltpu.VMEM((KB,), jnp.int32) for _ in range(4)]
            + [pltpu.VMEM((KB,), jnp.float32) for _ in range(4)]
            + [pltpu.VMEM((KB, HH), jnp.float32) for _ in range(4)]
            + [pltpu.SemaphoreType.DMA for _ in range(12)]
        ),
    )(hlpf, eidx, ewf)


def _dense1(x, rows_emb, W1, deg2):
    def body(x_ref, r_ref, w_ref, deg_ref, hl_ref, dinv_ref):
        h1 = jnp.dot(x_ref[...], w_ref[0:D, :], preferred_element_type=jnp.float32)
        h1 = h1 + jnp.dot(r_ref[...], w_ref[D:, :], preferred_element_type=jnp.float32)
        d = deg_ref[0, :, 0] + deg_ref[1, :, 0] + 1.0
        dinv = lax.rsqrt(d)
        h1 = h1 * dinv[:, None]
        hl_ref[0] = h1[:, :HH]
        hl_ref[1] = h1[:, HH:]
        dinv_ref[...] = dinv[:, None]

    return pl.pallas_call(
        body,
        grid=(N // RB,),
        in_specs=[
            pl.BlockSpec((RB, D), lambda i: (i, 0)),
            pl.BlockSpec((RB, D), lambda i: (i, 0)),
            pl.BlockSpec((H, H), lambda i: (0, 0)),
            pl.BlockSpec((NC, RB, 1), lambda i: (0, i, 0)),
        ],
        out_specs=[
            pl.BlockSpec((NC, RB, HH), lambda i: (0, i, 0)),
            pl.BlockSpec((RB, 1), lambda i: (i, 0)),
        ],
        out_shape=[
            jax.ShapeDtypeStruct((NC, N, HH), jnp.float32),
            jax.ShapeDtypeStruct((N, 1), jnp.float32),
        ],
    )(x, rows_emb, W1, deg2)


def _dense2(acc1, hlp1, dinv, b1, W2):
    def body(a_ref, h_ref, dinv_ref, b_ref, w_ref, out_ref):
        dv = dinv_ref[...]
        bl = b_ref[...]
        h0 = jnp.maximum(dv * (a_ref[0] + h_ref[0]) + bl[None, :HH], 0.0)
        h1 = jnp.maximum(dv * (a_ref[1] + h_ref[1]) + bl[None, HH:], 0.0)
        o = jnp.dot(h0, w_ref[0:HH, :], preferred_element_type=jnp.float32)
        o = o + jnp.dot(h1, w_ref[HH:, :], preferred_element_type=jnp.float32)
        o = o * dv
        out_ref[0] = o[:, :HH]
        out_ref[1] = o[:, HH:]

    return pl.pallas_call(
        body,
        grid=(N // RB,),
        in_specs=[
            pl.BlockSpec((NC, RB, HH), lambda i: (0, i, 0)),
            pl.BlockSpec((NC, RB, HH), lambda i: (0, i, 0)),
            pl.BlockSpec((RB, 1), lambda i: (i, 0)),
            pl.BlockSpec((H,), lambda i: (0,)),
            pl.BlockSpec((H, H), lambda i: (0, 0)),
        ],
        out_specs=pl.BlockSpec((NC, RB, HH), lambda i: (0, i, 0)),
        out_shape=jax.ShapeDtypeStruct((NC, N, HH), jnp.float32),
    )(acc1, hlp1, dinv, b1, W2)


def _pool(acc2, hlp2, dinv, b2, batch):
    nblk = N // RB

    def body(a_ref, h_ref, dinv_ref, b_ref, bt_ref, out_ref, acc, cnt):
        i = pl.program_id(0)
        dv = dinv_ref[...]
        bl = b_ref[...]
        h0 = jnp.maximum(dv * (a_ref[0] + h_ref[0]) + bl[None, :HH], 0.0)
        h1 = jnp.maximum(dv * (a_ref[1] + h_ref[1]) + bl[None, HH:], 0.0)
        bt = bt_ref[...]
        m = (bt == lax.broadcasted_iota(jnp.int32, (RB, G), 1)
             ).astype(jnp.float32)

        @pl.when(i == 0)
        def _():
            acc[...] = jnp.zeros_like(acc)
            cnt[...] = jnp.zeros_like(cnt)

        dn = (((0,), (0,)), ((), ()))
        p0 = lax.dot_general(m, h0, dn, preferred_element_type=jnp.float32)
        p1 = lax.dot_general(m, h1, dn, preferred_element_type=jnp.float32)
        acc[...] = acc[...] + jnp.concatenate([p0, p1], axis=1)
        cnt[...] = cnt[...] + jnp.sum(m, axis=0)[:, None]

        @pl.when(i == nblk - 1)
        def _():
            out_ref[...] = acc[...] / jnp.maximum(cnt[...], 1.0)

    return pl.pallas_call(
        body,
        grid=(nblk,),
        in_specs=[
            pl.BlockSpec((NC, RB, HH), lambda i: (0, i, 0)),
            pl.BlockSpec((NC, RB, HH), lambda i: (0, i, 0)),
            pl.BlockSpec((RB, 1), lambda i: (i, 0)),
            pl.BlockSpec((H,), lambda i: (0,)),
            pl.BlockSpec((RB, 1), lambda i: (i, 0)),
        ],
        out_specs=pl.BlockSpec((G, H), lambda i: (0, 0)),
        out_shape=jax.ShapeDtypeStruct((G, H), jnp.float32),
        scratch_shapes=[
            pltpu.VMEM((G, H), jnp.float32),
            pltpu.VMEM((G, 1), jnp.float32),
        ],
    )(acc2, hlp2, dinv, b2, batch)


def kernel(x, mapping, edge_index, edge_attr, batch, emb, W1, b1, W2, b2):
    row = edge_index[0]
    col = edge_index[1]
    padE = EP - E
    zi = jnp.zeros((padE,), jnp.int32)
    rowp = jnp.concatenate([row, zi])
    colp = jnp.concatenate([col, zi])
    wp = jnp.concatenate([edge_attr, jnp.zeros((padE,), jnp.float32)])
    mapping2 = jnp.concatenate(
        [mapping, jnp.zeros((NP - N,), jnp.int32)]).reshape(NP // 80, 80)

    eidx = jnp.concatenate(
        [rowp.reshape(-1, 64), colp.reshape(-1, 64)], axis=1).reshape(-1)

    rows_emb = _emb_gather(emb, mapping2)
    deg2 = _deg(colp, wp).reshape(NC, NP, 1)
    hlp1, dinv = _dense1(x, rows_emb, W1, deg2)
    acc1 = _agg(hlp1.reshape(NC * N, HH), eidx, wp)
    hlp2 = _dense2(acc1.reshape(NC, NP, HH), hlp1, dinv, b1, W2)
    acc2 = _agg(hlp2.reshape(NC * N, HH), eidx, wp)
    return _pool(acc2.reshape(NC, NP, HH), hlp2, dinv, b2, batch.reshape(N, 1))


# speedup vs baseline: 1.0387x; 1.0387x over previous
import functools

import jax
import jax.numpy as jnp
from jax import lax
from jax.experimental import pallas as pl
from jax.experimental.pallas import tpu as pltpu
from jax.experimental.pallas import tpu_sc as plsc

N = 10000
D = 128
H = 256
HH = 128
E = 320000
G = 64
NC = 2
NS = 16
L = 16

NP = 10240
EP = 327680
KE = 128
RB = 1000

_mesh = functools.partial(
    plsc.VectorSubcoreMesh,
    core_axis_name="c", subcore_axis_name="s", num_cores=NC, num_subcores=NS)


def _embdeg(emb, mapping2, colp, wp):
    BPW = NP // (NC * NS)
    CH = 80
    EPW = EP // (NC * NS)
    ZR = NP // NS

    def body(emb_hbm, idx_hbm, col_hbm, w_hbm, rows_out, deg_out,
             acc_sh, idx_v, rows_v, colv, wv, cidx, zbuf, gsem, dsem):
        c = lax.axis_index("c")
        s = lax.axis_index("s")
        wid_e = s * NC + c
        base = wid_e * BPW
        pltpu.sync_copy(idx_hbm.at[pl.ds(wid_e * (BPW // CH), BPW // CH)],
                        idx_v)
        for q in range(BPW // CH):
            pltpu.async_copy(emb_hbm.at[idx_v.at[q]],
                             rows_v.at[pl.ds(q * CH, CH)], gsem)
        wid_d = c * NS + s
        pltpu.sync_copy(col_hbm.at[pl.ds(wid_d * EPW, EPW)], colv)
        pltpu.sync_copy(w_hbm.at[pl.ds(wid_d * EPW, EPW)], wv)
        zeros = jnp.zeros((L,), jnp.float32)
        for r in range(L):
            zbuf[pl.ds(r * L, L)] = zeros
        def zloop(q, _):
            pltpu.sync_copy(zbuf, acc_sh.at[pl.ds(s * ZR + q * L * L, L * L)])
            return 0
        lax.fori_loop(0, ZR // (L * L), zloop, 0)
        plsc.subcore_barrier()

        def chunk(k, _):
            for j in range(KE // L):
                cidx[pl.ds(j * L, L)] = colv[pl.ds(k * KE + j * L, L)]
            pltpu.async_copy(wv.at[pl.ds(k * KE, KE)], acc_sh.at[cidx],
                             dsem, add=True).wait()
            return 0
        lax.fori_loop(0, EPW // KE, chunk, 0)
        plsc.subcore_barrier()
        pltpu.sync_copy(acc_sh.at[pl.ds(s * ZR, ZR)],
                        deg_out.at[pl.ds(c * NP + s * ZR, ZR)])
        for q in range(BPW // CH):
            pltpu.make_async_copy(emb_hbm.at[idx_v.at[q]],
                                  rows_v.at[pl.ds(q * CH, CH)], gsem).wait()
        pltpu.sync_copy(rows_v, rows_out.at[pl.ds(base, BPW)])

    return pl.kernel(
        body,
        out_type=[jax.ShapeDtypeStruct((NP, D), jnp.float32),
                  jax.ShapeDtypeStruct((NC * NP,), jnp.float32)],
        mesh=_mesh(),
        scratch_types=[
            pltpu.VMEM_SHARED((NP,), jnp.float32),
            pltpu.VMEM((BPW // CH, CH), jnp.int32),
            pltpu.VMEM((BPW, D), jnp.float32),
            pltpu.VMEM((EPW,), jnp.int32),
            pltpu.VMEM((EPW,), jnp.float32),
            pltpu.VMEM((KE,), jnp.int32),
            pltpu.VMEM((L * L,), jnp.float32),
            pltpu.SemaphoreType.DMA,
            pltpu.SemaphoreType.DMA,
        ],
    )(emb, mapping2, colp, wp)


def _agg(hlpf, rowp, colp, wp):
    EPT = EP // NS
    CS = 4096
    NCH = EPT // CS
    NBC = CS // KE
    ZR = NP // NS

    def body(hl_hbm, row_hbm, col_hbm, w_hbm, out_hbm,
             acc_sh, rowv, colv, wv,
             gidxA, cidxA, wblkA, rbufA, gidxB, cidxB, wblkB, rbufB,
             gsemA, gsemB, ssemA, ssemB):
        c = lax.axis_index("c")
        s = lax.axis_index("s")
        zeros = jnp.zeros((L,), jnp.float32)
        for r in range(L):
            for q in range(HH // L):
                rbufA[r, pl.ds(q * L, L)] = zeros
        def zloop(q, _):
            pltpu.sync_copy(rbufA.at[pl.ds(0, L)],
                            acc_sh.at[pl.ds(s * ZR + q * L, L)])
            return 0
        lax.fori_loop(0, ZR // L, zloop, 0)
        plsc.subcore_barrier()

        coff = c * N

        def build(gidx, cidx, wblk, eb):
            for j in range(KE // L):
                o = eb + j * L
                cidx[pl.ds(j * L, L)] = colv[pl.ds(o, L)]
                gidx[pl.ds(j * L, L)] = rowv[pl.ds(o, L)] + coff
                wblk[pl.ds(j * L, L)] = wv[pl.ds(o, L)]

        def build_gw(gidx, wblk, eb):
            for j in range(KE // L):
                o = eb + j * L
                gidx[pl.ds(j * L, L)] = rowv[pl.ds(o, L)] + coff
                wblk[pl.ds(j * L, L)] = wv[pl.ds(o, L)]

        def build_c(cidx, eb):
            for j in range(KE // L):
                cidx[pl.ds(j * L, L)] = colv[pl.ds(eb + j * L, L)]

        def scale(rbuf, wblk):
            def sj(j, _):
                wvv = wblk[pl.ds(j * L, L)]
                for t in range(L):
                    ns = jnp.broadcast_to(wvv[t], (L,))
                    e = j * L + t
                    for q in range(HH // L):
                        rbuf[e, pl.ds(q * L, L)] = rbuf[e, pl.ds(q * L, L)] * ns
                return 0
            lax.fori_loop(0, KE // L, sj, 0)

        def chunk(ch, _):
            ebase = s * EPT + ch * CS
            pltpu.sync_copy(row_hbm.at[pl.ds(ebase, CS)], rowv)
            pltpu.sync_copy(col_hbm.at[pl.ds(ebase, CS)], colv)
            pltpu.sync_copy(w_hbm.at[pl.ds(ebase, CS)], wv)
            build(gidxA, cidxA, wblkA, 0)
            pltpu.async_copy(hl_hbm.at[gidxA], rbufA, gsemA)

            def it(i, _):
                build(gidxB, cidxB, wblkB, (2 * i + 1) * KE)
                pltpu.async_copy(hl_hbm.at[gidxB], rbufB, gsemB)
                pltpu.make_async_copy(hl_hbm.at[gidxA], rbufA, gsemA).wait()
                scale(rbufA, wblkA)
                pltpu.async_copy(rbufA, acc_sh.at[cidxA], ssemA, add=True)
                build_gw(gidxA, wblkA, (2 * i + 2) * KE)
                pltpu.make_async_copy(rbufA, acc_sh.at[cidxA], ssemA).wait()
                build_c(cidxA, (2 * i + 2) * KE)
                pltpu.async_copy(hl_hbm.at[gidxA], rbufA, gsemA)
                pltpu.make_async_copy(hl_hbm.at[gidxB], rbufB, gsemB).wait()
                scale(rbufB, wblkB)
                pltpu.async_copy(rbufB, acc_sh.at[cidxB], ssemB, add=True)
                pltpu.make_async_copy(rbufB, acc_sh.at[cidxB], ssemB).wait()
                return 0
            lax.fori_loop(0, NBC // 2 - 1, it, 0)

            build(gidxB, cidxB, wblkB, (NBC - 1) * KE)
            pltpu.async_copy(hl_hbm.at[gidxB], rbufB, gsemB)
            pltpu.make_async_copy(hl_hbm.at[gidxA], rbufA, gsemA).wait()
            scale(rbufA, wblkA)
            pltpu.async_copy(rbufA, acc_sh.at[cidxA], ssemA, add=True)
            pltpu.make_async_copy(rbufA, acc_sh.at[cidxA], ssemA).wait()
            pltpu.make_async_copy(hl_hbm.at[gidxB], rbufB, gsemB).wait()
            scale(rbufB, wblkB)
            pltpu.async_copy(rbufB, acc_sh.at[cidxB], ssemB, add=True)
            pltpu.make_async_copy(rbufB, acc_sh.at[cidxB], ssemB).wait()
            return 0
        lax.fori_loop(0, NCH, chunk, 0)
        plsc.subcore_barrier()
        pltpu.sync_copy(acc_sh.at[pl.ds(s * ZR, ZR)],
                        out_hbm.at[pl.ds(c * NP + s * ZR, ZR)])

    return pl.kernel(
        body,
        out_type=jax.ShapeDtypeStruct((NC * NP, HH), jnp.float32),
        mesh=_mesh(),
        scratch_types=[
            pltpu.VMEM_SHARED((NP, HH), jnp.float32),
            pltpu.VMEM((CS,), jnp.int32),
            pltpu.VMEM((CS,), jnp.int32),
            pltpu.VMEM((CS,), jnp.float32),
            pltpu.VMEM((KE,), jnp.int32),
            pltpu.VMEM((KE,), jnp.int32),
            pltpu.VMEM((KE,), jnp.float32),
            pltpu.VMEM((KE, HH), jnp.float32),
            pltpu.VMEM((KE,), jnp.int32),
            pltpu.VMEM((KE,), jnp.int32),
            pltpu.VMEM((KE,), jnp.float32),
            pltpu.VMEM((KE, HH), jnp.float32),
            pltpu.SemaphoreType.DMA,
            pltpu.SemaphoreType.DMA,
            pltpu.SemaphoreType.DMA,
            pltpu.SemaphoreType.DMA,
        ],
    )(hlpf, rowp, colp, wp)


def _dense1(x, rows_emb, W1, deg2):
    def body(x_ref, r_ref, w_ref, deg_ref, hl_ref, dinv_ref):
        h1 = jnp.dot(x_ref[...], w_ref[0:D, :], preferred_element_type=jnp.float32)
        h1 = h1 + jnp.dot(r_ref[...], w_ref[D:, :], preferred_element_type=jnp.float32)
        d = deg_ref[0, :, 0] + deg_ref[1, :, 0] + 1.0
        dinv = lax.rsqrt(d)
        h1 = h1 * dinv[:, None]
        hl_ref[0] = h1[:, :HH]
        hl_ref[1] = h1[:, HH:]
        dinv_ref[...] = dinv[:, None]

    return pl.pallas_call(
        body,
        grid=(N // RB,),
        in_specs=[
            pl.BlockSpec((RB, D), lambda i: (i, 0)),
            pl.BlockSpec((RB, D), lambda i: (i, 0)),
            pl.BlockSpec((H, H), lambda i: (0, 0)),
            pl.BlockSpec((NC, RB, 1), lambda i: (0, i, 0)),
        ],
        out_specs=[
            pl.BlockSpec((NC, RB, HH), lambda i: (0, i, 0)),
            pl.BlockSpec((RB, 1), lambda i: (i, 0)),
        ],
        out_shape=[
            jax.ShapeDtypeStruct((NC, N, HH), jnp.float32),
            jax.ShapeDtypeStruct((N, 1), jnp.float32),
        ],
    )(x, rows_emb, W1, deg2)


def _dense2(acc1, hlp1, dinv, b1, W2):
    def body(a_ref, h_ref, dinv_ref, b_ref, w_ref, out_ref):
        dv = dinv_ref[...]
        bl = b_ref[...]
        h0 = jnp.maximum(dv * (a_ref[0] + h_ref[0]) + bl[None, :HH], 0.0)
        h1 = jnp.maximum(dv * (a_ref[1] + h_ref[1]) + bl[None, HH:], 0.0)
        o = jnp.dot(h0, w_ref[0:HH, :], preferred_element_type=jnp.float32)
        o = o + jnp.dot(h1, w_ref[HH:, :], preferred_element_type=jnp.float32)
        o = o * dv
        out_ref[0] = o[:, :HH]
        out_ref[1] = o[:, HH:]

    return pl.pallas_call(
        body,
        grid=(N // RB,),
        in_specs=[
            pl.BlockSpec((NC, RB, HH), lambda i: (0, i, 0)),
            pl.BlockSpec((NC, RB, HH), lambda i: (0, i, 0)),
            pl.BlockSpec((RB, 1), lambda i: (i, 0)),
            pl.BlockSpec((H,), lambda i: (0,)),
            pl.BlockSpec((H, H), lambda i: (0, 0)),
        ],
        out_specs=pl.BlockSpec((NC, RB, HH), lambda i: (0, i, 0)),
        out_shape=jax.ShapeDtypeStruct((NC, N, HH), jnp.float32),
    )(acc1, hlp1, dinv, b1, W2)


def _pool(acc2, hlp2, dinv, b2, batch):
    nblk = N // RB

    def body(a_ref, h_ref, dinv_ref, b_ref, bt_ref, out_ref, acc, cnt):
        i = pl.program_id(0)
        dv = dinv_ref[...]
        bl = b_ref[...]
        h0 = jnp.maximum(dv * (a_ref[0] + h_ref[0]) + bl[None, :HH], 0.0)
        h1 = jnp.maximum(dv * (a_ref[1] + h_ref[1]) + bl[None, HH:], 0.0)
        bt = bt_ref[...]
        m = (bt == lax.broadcasted_iota(jnp.int32, (RB, G), 1)
             ).astype(jnp.float32)

        @pl.when(i == 0)
        def _():
            acc[...] = jnp.zeros_like(acc)
            cnt[...] = jnp.zeros_like(cnt)

        dn = (((0,), (0,)), ((), ()))
        p0 = lax.dot_general(m, h0, dn, preferred_element_type=jnp.float32)
        p1 = lax.dot_general(m, h1, dn, preferred_element_type=jnp.float32)
        acc[...] = acc[...] + jnp.concatenate([p0, p1], axis=1)
        cnt[...] = cnt[...] + jnp.sum(m, axis=0)[:, None]

        @pl.when(i == nblk - 1)
        def _():
            out_ref[...] = acc[...] / jnp.maximum(cnt[...], 1.0)

    return pl.pallas_call(
        body,
        grid=(nblk,),
        in_specs=[
            pl.BlockSpec((NC, RB, HH), lambda i: (0, i, 0)),
            pl.BlockSpec((NC, RB, HH), lambda i: (0, i, 0)),
            pl.BlockSpec((RB, 1), lambda i: (i, 0)),
            pl.BlockSpec((H,), lambda i: (0,)),
            pl.BlockSpec((RB, 1), lambda i: (i, 0)),
        ],
        out_specs=pl.BlockSpec((G, H), lambda i: (0, 0)),
        out_shape=jax.ShapeDtypeStruct((G, H), jnp.float32),
        scratch_shapes=[
            pltpu.VMEM((G, H), jnp.float32),
            pltpu.VMEM((G, 1), jnp.float32),
        ],
    )(acc2, hlp2, dinv, b2, batch)


def kernel(x, mapping, edge_index, edge_attr, batch, emb, W1, b1, W2, b2):
    row = edge_index[0]
    col = edge_index[1]
    padE = EP - E
    zi = jnp.zeros((padE,), jnp.int32)
    rowp = jnp.concatenate([row, zi])
    colp = jnp.concatenate([col, zi])
    wp = jnp.concatenate([edge_attr, jnp.zeros((padE,), jnp.float32)])
    mapping2 = jnp.concatenate(
        [mapping, jnp.zeros((NP - N,), jnp.int32)]).reshape(NP // 80, 80)

    rows_emb, deg = _embdeg(emb, mapping2, colp, wp)
    deg2 = deg.reshape(NC, NP, 1)
    hlp1, dinv = _dense1(x, rows_emb, W1, deg2)
    acc1 = _agg(hlp1.reshape(NC * N, HH), rowp, colp, wp)
    hlp2 = _dense2(acc1.reshape(NC, NP, HH), hlp1, dinv, b1, W2)
    acc2 = _agg(hlp2.reshape(NC * N, HH), rowp, colp, wp)
    return _pool(acc2.reshape(NC, NP, HH), hlp2, dinv, b2, batch.reshape(N, 1))
